# Initial kernel scaffold; baseline (speedup 1.0000x reference)
import jax
import jax.numpy as jnp
from jax.experimental import pallas as pl


def kernel(input, W):
    raise NotImplementedError("write your pallas kernel here")


# speedup vs baseline: 1.3797x; 1.3797x over previous
import functools

import jax
import jax.numpy as jnp
from jax.experimental import pallas as pl
from jax.experimental.pallas import tpu as pltpu

_E = 64
_TOPK = 8
_D = 4096
_S = 8192
_BS = 256
_NB = _S // _BS
_BS2 = 1024
_NB2 = _S // _BS2
_EPS = float(jnp.finfo(jnp.float32).eps)


def _cumsum_rows(x):
    n, e = x.shape
    shift = 1
    while shift < n:
        x = x + jnp.concatenate(
            [jnp.zeros((shift, e), x.dtype), x[: n - shift]], axis=0)
        shift *= 2
    return x


def _router_body(x_ref, wt_ref, gates_ref, idx_ref, locw_ref, counts_ref,
                 me_ref, ce_ref, loss_ref, carry_ref, me_acc, ce_acc):
    b = pl.program_id(0)

    @pl.when(b == 0)
    def _():
        carry_ref[...] = jnp.zeros_like(carry_ref)
        me_acc[...] = jnp.zeros_like(me_acc)
        ce_acc[...] = jnp.zeros_like(ce_acc)

    logits = jnp.dot(x_ref[...], wt_ref[...],
                     preferred_element_type=jnp.float32)
    iota_e = jax.lax.broadcasted_iota(jnp.int32, (_BS, _E), 1)

    cur = logits
    val_cols, idx_cols, sels = [], [], []
    for _ in range(_TOPK):
        m = jnp.max(cur, axis=1, keepdims=True)
        ik = jnp.min(jnp.where(cur == m, iota_e, _E), axis=1, keepdims=True)
        sel = iota_e == ik
        cur = jnp.where(sel, -jnp.inf, cur)
        val_cols.append(m)
        idx_cols.append(ik)
        sels.append(sel)

    maxv = val_cols[0]
    expl = jnp.exp(logits - maxv)
    sumexp = jnp.sum(expl, axis=1, keepdims=True)
    gate_cols = [jnp.exp(v - maxv) / sumexp for v in val_cols]
    denom = jnp.maximum(functools.reduce(lambda a, c: a + c, gate_cols), _EPS)
    gates_ref[...] = jnp.concatenate(gate_cols, axis=1) / denom
    idx_ref[...] = jnp.concatenate(idx_cols, axis=1)

    locw_cols, cnt_rows = [], []
    for k in range(_TOPK):
        selk = sels[k].astype(jnp.int32)
        csum = _cumsum_rows(selk)
        carry_k = carry_ref[k:k + 1, :]
        loc = jnp.sum((csum - 1 + carry_k) * selk, axis=1, keepdims=True)
        locw_cols.append(loc)
        cnt_rows.append(jnp.sum(selk, axis=0, keepdims=True))
    locw_ref[...] = jnp.concatenate(locw_cols, axis=1)
    carry_ref[...] = carry_ref[...] + jnp.concatenate(cnt_rows, axis=0)

    me_acc[...] = me_acc[...] + jnp.sum(expl / sumexp, axis=0, keepdims=True)
    ce_acc[...] = ce_acc[...] + jnp.sum(sels[0].astype(jnp.float32), axis=0,
                                        keepdims=True)

    @pl.when(b == _NB - 1)
    def _():
        me_ref[...] = me_acc[...]
        ce_ref[...] = ce_acc[...]
        counts_ref[...] = carry_ref[...]
        loss_ref[...] = (jnp.sum(me_acc[...] * ce_acc[...], keepdims=True)
                         * (_E / (_S * _S)))


def _offset_body(counts_ref, idx_ref, locw_ref, loc_ref):
    counts = counts_ref[...]
    iota_e = jax.lax.broadcasted_iota(jnp.int32, (_BS2, _E), 1)
    acc = jnp.zeros((1, _E), jnp.int32)
    cols = [locw_ref[:, 0:1]]
    for k in range(1, _TOPK):
        acc = acc + counts[k - 1:k, :]
        idx_k = idx_ref[:, k:k + 1]
        off_tok = jnp.sum(jnp.where(iota_e == idx_k, acc, 0), axis=1,
                          keepdims=True)
        cols.append(locw_ref[:, k:k + 1] + off_tok)
    loc_ref[...] = jnp.concatenate(cols, axis=1)


def _run(x, wt, interpret=False):
    gates, idx, locw, counts, me, ce, loss = pl.pallas_call(
        _router_body,
        grid=(_NB,),
        in_specs=[
            pl.BlockSpec((_BS, _D), lambda i: (i, 0)),
            pl.BlockSpec((_D, _E), lambda i: (0, 0)),
        ],
        out_specs=[
            pl.BlockSpec((_BS, _TOPK), lambda i: (i, 0)),
            pl.BlockSpec((_BS, _TOPK), lambda i: (i, 0)),
            pl.BlockSpec((_BS, _TOPK), lambda i: (i, 0)),
            pl.BlockSpec((_TOPK, _E), lambda i: (0, 0)),
            pl.BlockSpec((1, _E), lambda i: (0, 0)),
            pl.BlockSpec((1, _E), lambda i: (0, 0)),
            pl.BlockSpec((1, 1), lambda i: (0, 0)),
        ],
        out_shape=[
            jax.ShapeDtypeStruct((_S, _TOPK), jnp.float32),
            jax.ShapeDtypeStruct((_S, _TOPK), jnp.int32),
            jax.ShapeDtypeStruct((_S, _TOPK), jnp.int32),
            jax.ShapeDtypeStruct((_TOPK, _E), jnp.int32),
            jax.ShapeDtypeStruct((1, _E), jnp.float32),
            jax.ShapeDtypeStruct((1, _E), jnp.float32),
            jax.ShapeDtypeStruct((1, 1), jnp.float32),
        ],
        scratch_shapes=[
            pltpu.VMEM((_TOPK, _E), jnp.int32),
            pltpu.VMEM((1, _E), jnp.float32),
            pltpu.VMEM((1, _E), jnp.float32),
        ],
        interpret=interpret,
    )(x, wt)
    loc = pl.pallas_call(
        _offset_body,
        grid=(_NB2,),
        in_specs=[
            pl.BlockSpec((_TOPK, _E), lambda i: (0, 0)),
            pl.BlockSpec((_BS2, _TOPK), lambda i: (i, 0)),
            pl.BlockSpec((_BS2, _TOPK), lambda i: (i, 0)),
        ],
        out_specs=pl.BlockSpec((_BS2, _TOPK), lambda i: (i, 0)),
        out_shape=jax.ShapeDtypeStruct((_S, _TOPK), jnp.int32),
        interpret=interpret,
    )(counts, idx, locw)
    return gates, idx, loc, loss


def kernel(input, W):
    gates, idx, loc, loss = _run(input, W.T)
    return (loss[0, 0],
            tuple(gates[:, k] for k in range(_TOPK)),
            tuple(idx[:, k] for k in range(_TOPK)),
            tuple(loc[:, k] for k in range(_TOPK)))


# speedup vs baseline: 4.1943x; 3.0399x over previous
import jax
import jax.numpy as jnp
from jax.experimental import pallas as pl
from jax.experimental.pallas import tpu as pltpu

_E = 64
_TOPK = 8
_D = 4096
_S = 8192
_BS = 256
_NB = _S // _BS
_BS2 = 2048
_NB2 = _S // _BS2
_EPS = float(jnp.finfo(jnp.float32).eps)


def _router_body(x_ref, w_ref, triu_ref, *refs):
    gates_refs = refs[0:_TOPK]
    idx_refs = refs[_TOPK:2 * _TOPK]
    locw_ref, counts_ref, loss_ref, carry_ref, me_acc = refs[2 * _TOPK:]
    b = pl.program_id(0)

    @pl.when(b == 0)
    def _():
        carry_ref[...] = jnp.zeros_like(carry_ref)
        me_acc[...] = jnp.zeros_like(me_acc)

    logits = jax.lax.dot_general(
        w_ref[...], x_ref[...], (((1,), (1,)), ((), ())),
        preferred_element_type=jnp.float32)

    iota_e = jax.lax.broadcasted_iota(jnp.int32, (_E, _BS), 0)

    cur = logits
    val_rows, idx_rows, sels = [], [], []
    for _ in range(_TOPK):
        m = jnp.max(cur, axis=0, keepdims=True)
        ik = jnp.min(jnp.where(cur == m, iota_e, _E), axis=0, keepdims=True)
        sel = iota_e == ik
        cur = jnp.where(sel, -jnp.inf, cur)
        val_rows.append(m)
        idx_rows.append(ik)
        sels.append(sel)

    maxv = val_rows[0]
    expl = jnp.exp(logits - maxv)
    inv = 1.0 / jnp.sum(expl, axis=0, keepdims=True)
    gate_rows = [jnp.exp(v - maxv) * inv for v in val_rows]
    denom = gate_rows[0]
    for g in gate_rows[1:]:
        denom = denom + g
    inv_denom = 1.0 / jnp.maximum(denom, _EPS)
    for k in range(_TOPK):
        gates_refs[k][...] = gate_rows[k] * inv_denom
        idx_refs[k][...] = idx_rows[k]

    locw_rows, cnt_cols = [], []
    for k in range(_TOPK):
        sel = sels[k]
        csum = jax.lax.dot_general(
            sel.astype(jnp.bfloat16), triu_ref[...], (((1,), (0,)), ((), ())),
            preferred_element_type=jnp.float32)
        carry_k = carry_ref[:, k:k + 1]
        loc_f = jnp.sum(jnp.where(sel, csum - 1.0 + carry_k, 0.0),
                        axis=0, keepdims=True)
        locw_rows.append(loc_f.astype(jnp.int32))
        cnt_cols.append(csum[:, _BS - 1:_BS])
    locw_ref[...] = jnp.concatenate(locw_rows, axis=0)
    carry_ref[...] = carry_ref[...] + jnp.concatenate(cnt_cols, axis=1)

    me_acc[...] = me_acc[...] + jnp.sum(expl * inv, axis=1, keepdims=True)

    @pl.when(b == _NB - 1)
    def _():
        counts_ref[...] = carry_ref[...]
        loss_ref[...] = (jnp.sum(me_acc[...] * carry_ref[:, 0:1],
                                 keepdims=True) * (_E / (_S * _S)))


def _offset_body(counts_ref, locw_ref, *refs):
    idx_refs = refs[0:_TOPK]
    loc_refs = refs[_TOPK:]
    counts = counts_ref[...]
    iota_e = jax.lax.broadcasted_iota(jnp.int32, (_E, _BS2), 0)
    loc_refs[0][...] = locw_ref[0:1, :]
    acc = counts[:, 0:1]
    for k in range(1, _TOPK):
        eq = iota_e == idx_refs[k][...]
        off_tok = jnp.sum(jnp.where(eq, acc, 0.0), axis=0, keepdims=True)
        loc_refs[k][...] = locw_ref[k:k + 1, :] + off_tok.astype(jnp.int32)
        if k < _TOPK - 1:
            acc = acc + counts[:, k:k + 1]


def _run(x, W, interpret=False):
    triu = jnp.triu(jnp.ones((_BS, _BS), jnp.bfloat16))
    row_spec = pl.BlockSpec((1, _BS), lambda i: (0, i))
    outs = pl.pallas_call(
        _router_body,
        grid=(_NB,),
        in_specs=[
            pl.BlockSpec((_BS, _D), lambda i: (i, 0)),
            pl.BlockSpec((_E, _D), lambda i: (0, 0)),
            pl.BlockSpec((_BS, _BS), lambda i: (0, 0)),
        ],
        out_specs=(
            [row_spec] * (2 * _TOPK)
            + [pl.BlockSpec((_TOPK, _BS), lambda i: (0, i)),
               pl.BlockSpec((_E, _TOPK), lambda i: (0, 0)),
               pl.BlockSpec((1, 1), lambda i: (0, 0))]
        ),
        out_shape=(
            [jax.ShapeDtypeStruct((1, _S), jnp.float32)] * _TOPK
            + [jax.ShapeDtypeStruct((1, _S), jnp.int32)] * _TOPK
            + [jax.ShapeDtypeStruct((_TOPK, _S), jnp.int32),
               jax.ShapeDtypeStruct((_E, _TOPK), jnp.float32),
               jax.ShapeDtypeStruct((1, 1), jnp.float32)]
        ),
        scratch_shapes=[
            pltpu.VMEM((_E, _TOPK), jnp.float32),
            pltpu.VMEM((_E, 1), jnp.float32),
        ],
        interpret=interpret,
    )(x, W, triu)
    gates = outs[0:_TOPK]
    idxs = outs[_TOPK:2 * _TOPK]
    locw, counts, loss = outs[2 * _TOPK:]
    row_spec2 = pl.BlockSpec((1, _BS2), lambda i: (0, i))
    locs = pl.pallas_call(
        _offset_body,
        grid=(_NB2,),
        in_specs=(
            [pl.BlockSpec((_E, _TOPK), lambda i: (0, 0)),
             pl.BlockSpec((_TOPK, _BS2), lambda i: (0, i))]
            + [row_spec2] * _TOPK
        ),
        out_specs=[row_spec2] * _TOPK,
        out_shape=[jax.ShapeDtypeStruct((1, _S), jnp.int32)] * _TOPK,
        interpret=interpret,
    )(counts, locw, *idxs)
    return gates, idxs, locs, loss


def kernel(input, W):
    gates, idxs, locs, loss = _run(input, W)
    return (jnp.reshape(loss, ()),
            tuple(jnp.reshape(g, (_S,)) for g in gates),
            tuple(jnp.reshape(i, (_S,)) for i in idxs),
            tuple(jnp.reshape(l, (_S,)) for l in locs))


# speedup vs baseline: 4.2916x; 1.0232x over previous
import jax
import jax.numpy as jnp
from jax.experimental import pallas as pl
from jax.experimental.pallas import tpu as pltpu

_E = 64
_TOPK = 8
_D = 4096
_S = 8192
_BS = 256
_NB = _S // _BS
_BS2 = 2048
_NB2 = _S // _BS2
_EPS = float(jnp.finfo(jnp.float32).eps)


def _router_body(x_ref, w_ref, triu_ref, *refs):
    gates_refs = refs[0:_TOPK]
    idx_refs = refs[_TOPK:2 * _TOPK]
    locw_ref, counts_ref, loss_ref, carry_ref, me_acc = refs[2 * _TOPK:]
    b = pl.program_id(0)

    @pl.when(b == 0)
    def _():
        carry_ref[...] = jnp.zeros_like(carry_ref)
        me_acc[...] = jnp.zeros_like(me_acc)

    logits = jax.lax.dot_general(
        w_ref[...], x_ref[...], (((1,), (1,)), ((), ())),
        preferred_element_type=jnp.float32)

    iota_e = jax.lax.broadcasted_iota(jnp.int32, (_E, _BS), 0)

    cur = logits
    val_rows, idx_rows, sels = [], [], []
    for _ in range(_TOPK):
        m = jnp.max(cur, axis=0, keepdims=True)
        ik = jnp.min(jnp.where(cur == m, iota_e, _E), axis=0, keepdims=True)
        sel = iota_e == ik
        cur = jnp.where(sel, -jnp.inf, cur)
        val_rows.append(m)
        idx_rows.append(ik)
        sels.append(sel)

    maxv = val_rows[0]
    expl = jnp.exp(logits - maxv)
    inv = 1.0 / jnp.sum(expl, axis=0, keepdims=True)
    gate_rows = [jnp.exp(v - maxv) * inv for v in val_rows]
    denom = gate_rows[0]
    for g in gate_rows[1:]:
        denom = denom + g
    inv_denom = 1.0 / jnp.maximum(denom, _EPS)
    for k in range(_TOPK):
        gates_refs[k][...] = gate_rows[k] * inv_denom
        idx_refs[k][...] = idx_rows[k]

    sel_all = jnp.concatenate(sels, axis=0).astype(jnp.bfloat16)
    csum_all = jax.lax.dot_general(
        sel_all, triu_ref[...], (((1,), (0,)), ((), ())),
        preferred_element_type=jnp.float32)
    locw_rows, cnt_cols = [], []
    for k in range(_TOPK):
        sel = sels[k]
        csum = csum_all[k * _E:(k + 1) * _E, :]
        carry_k = carry_ref[:, k:k + 1]
        loc_f = jnp.sum(jnp.where(sel, csum - 1.0 + carry_k, 0.0),
                        axis=0, keepdims=True)
        locw_rows.append(loc_f.astype(jnp.int32))
        cnt_cols.append(csum[:, _BS - 1:_BS])
    locw_ref[...] = jnp.concatenate(locw_rows, axis=0)
    carry_ref[...] = carry_ref[...] + jnp.concatenate(cnt_cols, axis=1)

    me_acc[...] = me_acc[...] + jnp.sum(expl * inv, axis=1, keepdims=True)

    @pl.when(b == _NB - 1)
    def _():
        counts_ref[...] = carry_ref[...]
        loss_ref[...] = (jnp.sum(me_acc[...] * carry_ref[:, 0:1],
                                 keepdims=True) * (_E / (_S * _S)))


def _offset_body(counts_ref, locw_ref, *refs):
    idx_refs = refs[0:_TOPK]
    loc_refs = refs[_TOPK:]
    counts = counts_ref[...]
    iota_e = jax.lax.broadcasted_iota(jnp.int32, (_E, _BS2), 0)
    loc_refs[0][...] = locw_ref[0:1, :]
    acc = counts[:, 0:1]
    for k in range(1, _TOPK):
        eq = iota_e == idx_refs[k][...]
        off_tok = jnp.sum(jnp.where(eq, acc, 0.0), axis=0, keepdims=True)
        loc_refs[k][...] = locw_ref[k:k + 1, :] + off_tok.astype(jnp.int32)
        if k < _TOPK - 1:
            acc = acc + counts[:, k:k + 1]


def _run(x, W, interpret=False):
    triu = jnp.triu(jnp.ones((_BS, _BS), jnp.bfloat16))
    row_spec = pl.BlockSpec((1, _BS), lambda i: (0, i))
    outs = pl.pallas_call(
        _router_body,
        grid=(_NB,),
        in_specs=[
            pl.BlockSpec((_BS, _D), lambda i: (i, 0)),
            pl.BlockSpec((_E, _D), lambda i: (0, 0)),
            pl.BlockSpec((_BS, _BS), lambda i: (0, 0)),
        ],
        out_specs=(
            [row_spec] * (2 * _TOPK)
            + [pl.BlockSpec((_TOPK, _BS), lambda i: (0, i)),
               pl.BlockSpec((_E, _TOPK), lambda i: (0, 0)),
               pl.BlockSpec((1, 1), lambda i: (0, 0))]
        ),
        out_shape=(
            [jax.ShapeDtypeStruct((1, _S), jnp.float32)] * _TOPK
            + [jax.ShapeDtypeStruct((1, _S), jnp.int32)] * _TOPK
            + [jax.ShapeDtypeStruct((_TOPK, _S), jnp.int32),
               jax.ShapeDtypeStruct((_E, _TOPK), jnp.float32),
               jax.ShapeDtypeStruct((1, 1), jnp.float32)]
        ),
        scratch_shapes=[
            pltpu.VMEM((_E, _TOPK), jnp.float32),
            pltpu.VMEM((_E, 1), jnp.float32),
        ],
        interpret=interpret,
    )(x, W, triu)
    gates = outs[0:_TOPK]
    idxs = outs[_TOPK:2 * _TOPK]
    locw, counts, loss = outs[2 * _TOPK:]
    row_spec2 = pl.BlockSpec((1, _BS2), lambda i: (0, i))
    locs = pl.pallas_call(
        _offset_body,
        grid=(_NB2,),
        in_specs=(
            [pl.BlockSpec((_E, _TOPK), lambda i: (0, 0)),
             pl.BlockSpec((_TOPK, _BS2), lambda i: (0, i))]
            + [row_spec2] * _TOPK
        ),
        out_specs=[row_spec2] * _TOPK,
        out_shape=[jax.ShapeDtypeStruct((1, _S), jnp.int32)] * _TOPK,
        interpret=interpret,
    )(counts, locw, *idxs)
    return gates, idxs, locs, loss


def kernel(input, W):
    gates, idxs, locs, loss = _run(input, W)
    return (jnp.reshape(loss, ()),
            tuple(jnp.reshape(g, (_S,)) for g in gates),
            tuple(jnp.reshape(i, (_S,)) for i in idxs),
            tuple(jnp.reshape(l, (_S,)) for l in locs))


# speedup vs baseline: 5.0735x; 1.1822x over previous
import jax
import jax.numpy as jnp
from jax.experimental import pallas as pl
from jax.experimental.pallas import tpu as pltpu

_E = 64
_TOPK = 8
_D = 4096
_S = 8192
_BS = 512
_NB = _S // _BS
_BS2 = 2048
_NB2 = _S // _BS2
_EPS = float(jnp.finfo(jnp.float32).eps)


def _router_body(x_ref, w_ref, triu_ref, *refs):
    gates_refs = refs[0:_TOPK]
    idx_refs = refs[_TOPK:2 * _TOPK]
    locw_ref, counts_ref, loss_ref, carry_ref, me_acc = refs[2 * _TOPK:]
    b = pl.program_id(0)

    @pl.when(b == 0)
    def _():
        carry_ref[...] = jnp.zeros_like(carry_ref)
        me_acc[...] = jnp.zeros_like(me_acc)

    logits = jax.lax.dot_general(
        w_ref[...], x_ref[...], (((1,), (1,)), ((), ())),
        preferred_element_type=jnp.float32)

    iota_e = jax.lax.broadcasted_iota(jnp.int32, (_E, _BS), 0)

    cur = logits
    val_rows, idx_rows, sels = [], [], []
    for _ in range(_TOPK):
        m = jnp.max(cur, axis=0, keepdims=True)
        ik = jnp.min(jnp.where(cur == m, iota_e, _E), axis=0, keepdims=True)
        sel = iota_e == ik
        cur = jnp.where(sel, -jnp.inf, cur)
        val_rows.append(m)
        idx_rows.append(ik)
        sels.append(sel)

    maxv = val_rows[0]
    expl = jnp.exp(logits - maxv)
    inv = 1.0 / jnp.sum(expl, axis=0, keepdims=True)
    gate_rows = [jnp.exp(v - maxv) * inv for v in val_rows]
    denom = gate_rows[0]
    for g in gate_rows[1:]:
        denom = denom + g
    inv_denom = 1.0 / jnp.maximum(denom, _EPS)
    for k in range(_TOPK):
        gates_refs[k][...] = gate_rows[k] * inv_denom
        idx_refs[k][...] = idx_rows[k]

    sel_all = jnp.concatenate(sels, axis=0).astype(jnp.bfloat16)
    csum_all = jax.lax.dot_general(
        sel_all, triu_ref[...], (((1,), (0,)), ((), ())),
        preferred_element_type=jnp.float32)
    locw_rows, cnt_cols = [], []
    for k in range(_TOPK):
        sel = sels[k]
        csum = csum_all[k * _E:(k + 1) * _E, :]
        carry_k = carry_ref[:, k:k + 1]
        loc_f = jnp.sum(jnp.where(sel, csum - 1.0 + carry_k, 0.0),
                        axis=0, keepdims=True)
        locw_rows.append(loc_f.astype(jnp.int32))
        cnt_cols.append(csum[:, _BS - 1:_BS])
    locw_ref[...] = jnp.concatenate(locw_rows, axis=0)
    carry_ref[...] = carry_ref[...] + jnp.concatenate(cnt_cols, axis=1)

    me_acc[...] = me_acc[...] + jnp.sum(expl * inv, axis=1, keepdims=True)

    @pl.when(b == _NB - 1)
    def _():
        counts_ref[...] = carry_ref[...]
        loss_ref[...] = (jnp.sum(me_acc[...] * carry_ref[:, 0:1],
                                 keepdims=True) * (_E / (_S * _S)))


def _offset_body(counts_ref, locw_ref, *refs):
    idx_refs = refs[0:_TOPK]
    loc_refs = refs[_TOPK:]
    counts = counts_ref[...]
    iota_e = jax.lax.broadcasted_iota(jnp.int32, (_E, _BS2), 0)
    loc_refs[0][...] = locw_ref[0:1, :]
    acc = counts[:, 0:1]
    for k in range(1, _TOPK):
        eq = iota_e == idx_refs[k][...]
        off_tok = jnp.sum(jnp.where(eq, acc, 0.0), axis=0, keepdims=True)
        loc_refs[k][...] = locw_ref[k:k + 1, :] + off_tok.astype(jnp.int32)
        if k < _TOPK - 1:
            acc = acc + counts[:, k:k + 1]


def _run(x, W, interpret=False):
    triu = jnp.triu(jnp.ones((_BS, _BS), jnp.bfloat16))
    row_spec = pl.BlockSpec((1, _BS), lambda i: (0, i))
    outs = pl.pallas_call(
        _router_body,
        grid=(_NB,),
        in_specs=[
            pl.BlockSpec((_BS, _D), lambda i: (i, 0)),
            pl.BlockSpec((_E, _D), lambda i: (0, 0)),
            pl.BlockSpec((_BS, _BS), lambda i: (0, 0)),
        ],
        out_specs=(
            [row_spec] * (2 * _TOPK)
            + [pl.BlockSpec((_TOPK, _BS), lambda i: (0, i)),
               pl.BlockSpec((_E, _TOPK), lambda i: (0, 0)),
               pl.BlockSpec((1, 1), lambda i: (0, 0))]
        ),
        out_shape=(
            [jax.ShapeDtypeStruct((1, _S), jnp.float32)] * _TOPK
            + [jax.ShapeDtypeStruct((1, _S), jnp.int32)] * _TOPK
            + [jax.ShapeDtypeStruct((_TOPK, _S), jnp.int32),
               jax.ShapeDtypeStruct((_E, _TOPK), jnp.float32),
               jax.ShapeDtypeStruct((1, 1), jnp.float32)]
        ),
        scratch_shapes=[
            pltpu.VMEM((_E, _TOPK), jnp.float32),
            pltpu.VMEM((_E, 1), jnp.float32),
        ],
        interpret=interpret,
    )(x, W, triu)
    gates = outs[0:_TOPK]
    idxs = outs[_TOPK:2 * _TOPK]
    locw, counts, loss = outs[2 * _TOPK:]
    row_spec2 = pl.BlockSpec((1, _BS2), lambda i: (0, i))
    locs = pl.pallas_call(
        _offset_body,
        grid=(_NB2,),
        in_specs=(
            [pl.BlockSpec((_E, _TOPK), lambda i: (0, 0)),
             pl.BlockSpec((_TOPK, _BS2), lambda i: (0, i))]
            + [row_spec2] * _TOPK
        ),
        out_specs=[row_spec2] * _TOPK,
        out_shape=[jax.ShapeDtypeStruct((1, _S), jnp.int32)] * _TOPK,
        interpret=interpret,
    )(counts, locw, *idxs)
    return gates, idxs, locs, loss


def kernel(input, W):
    gates, idxs, locs, loss = _run(input, W)
    return (jnp.reshape(loss, ()),
            tuple(jnp.reshape(g, (_S,)) for g in gates),
            tuple(jnp.reshape(i, (_S,)) for i in idxs),
            tuple(jnp.reshape(l, (_S,)) for l in locs))


# speedup vs baseline: 5.1927x; 1.0235x over previous
import jax
import jax.numpy as jnp
from jax.experimental import pallas as pl
from jax.experimental.pallas import tpu as pltpu

_E = 64
_TOPK = 8
_D = 4096
_S = 8192
_BS = 1024
_NB = _S // _BS
_BS2 = 2048
_NB2 = _S // _BS2
_EPS = float(jnp.finfo(jnp.float32).eps)


def _router_body(x_ref, w_ref, triu_ref, *refs):
    gates_refs = refs[0:_TOPK]
    idx_refs = refs[_TOPK:2 * _TOPK]
    locw_ref, counts_ref, loss_ref, carry_ref, me_acc = refs[2 * _TOPK:]
    b = pl.program_id(0)

    @pl.when(b == 0)
    def _():
        carry_ref[...] = jnp.zeros_like(carry_ref)
        me_acc[...] = jnp.zeros_like(me_acc)

    logits = jax.lax.dot_general(
        w_ref[...], x_ref[...], (((1,), (1,)), ((), ())),
        preferred_element_type=jnp.float32)

    iota_e = jax.lax.broadcasted_iota(jnp.int32, (_E, _BS), 0)

    cur = logits
    val_rows, idx_rows, sels = [], [], []
    for _ in range(_TOPK):
        m = jnp.max(cur, axis=0, keepdims=True)
        ik = jnp.min(jnp.where(cur == m, iota_e, _E), axis=0, keepdims=True)
        sel = iota_e == ik
        cur = jnp.where(sel, -jnp.inf, cur)
        val_rows.append(m)
        idx_rows.append(ik)
        sels.append(sel)

    maxv = val_rows[0]
    expl = jnp.exp(logits - maxv)
    inv = 1.0 / jnp.sum(expl, axis=0, keepdims=True)
    gate_rows = [jnp.exp(v - maxv) * inv for v in val_rows]
    denom = gate_rows[0]
    for g in gate_rows[1:]:
        denom = denom + g
    inv_denom = 1.0 / jnp.maximum(denom, _EPS)
    for k in range(_TOPK):
        gates_refs[k][...] = gate_rows[k] * inv_denom
        idx_refs[k][...] = idx_rows[k]

    sel_all = jnp.concatenate(sels, axis=0).astype(jnp.bfloat16)
    csum_all = jax.lax.dot_general(
        sel_all, triu_ref[...], (((1,), (0,)), ((), ())),
        preferred_element_type=jnp.float32)
    locw_rows, cnt_cols = [], []
    for k in range(_TOPK):
        sel = sels[k]
        csum = csum_all[k * _E:(k + 1) * _E, :]
        carry_k = carry_ref[:, k:k + 1]
        loc_f = jnp.sum(jnp.where(sel, csum - 1.0 + carry_k, 0.0),
                        axis=0, keepdims=True)
        locw_rows.append(loc_f.astype(jnp.int32))
        cnt_cols.append(csum[:, _BS - 1:_BS])
    locw_ref[...] = jnp.concatenate(locw_rows, axis=0)
    carry_ref[...] = carry_ref[...] + jnp.concatenate(cnt_cols, axis=1)

    me_acc[...] = me_acc[...] + jnp.sum(expl * inv, axis=1, keepdims=True)

    @pl.when(b == _NB - 1)
    def _():
        counts_ref[...] = carry_ref[...]
        loss_ref[...] = (jnp.sum(me_acc[...] * carry_ref[:, 0:1],
                                 keepdims=True) * (_E / (_S * _S)))


def _offset_body(counts_ref, locw_ref, *refs):
    idx_refs = refs[0:_TOPK]
    loc_refs = refs[_TOPK:]
    counts = counts_ref[...]
    iota_e = jax.lax.broadcasted_iota(jnp.int32, (_E, _BS2), 0)
    loc_refs[0][...] = locw_ref[0:1, :]
    acc = counts[:, 0:1]
    for k in range(1, _TOPK):
        eq = iota_e == idx_refs[k][...]
        off_tok = jnp.sum(jnp.where(eq, acc, 0.0), axis=0, keepdims=True)
        loc_refs[k][...] = locw_ref[k:k + 1, :] + off_tok.astype(jnp.int32)
        if k < _TOPK - 1:
            acc = acc + counts[:, k:k + 1]


def _run(x, W, interpret=False):
    triu = jnp.triu(jnp.ones((_BS, _BS), jnp.bfloat16))
    row_spec = pl.BlockSpec((1, _BS), lambda i: (0, i))
    outs = pl.pallas_call(
        _router_body,
        grid=(_NB,),
        in_specs=[
            pl.BlockSpec((_BS, _D), lambda i: (i, 0)),
            pl.BlockSpec((_E, _D), lambda i: (0, 0)),
            pl.BlockSpec((_BS, _BS), lambda i: (0, 0)),
        ],
        out_specs=(
            [row_spec] * (2 * _TOPK)
            + [pl.BlockSpec((_TOPK, _BS), lambda i: (0, i)),
               pl.BlockSpec((_E, _TOPK), lambda i: (0, 0)),
               pl.BlockSpec((1, 1), lambda i: (0, 0))]
        ),
        out_shape=(
            [jax.ShapeDtypeStruct((1, _S), jnp.float32)] * _TOPK
            + [jax.ShapeDtypeStruct((1, _S), jnp.int32)] * _TOPK
            + [jax.ShapeDtypeStruct((_TOPK, _S), jnp.int32),
               jax.ShapeDtypeStruct((_E, _TOPK), jnp.float32),
               jax.ShapeDtypeStruct((1, 1), jnp.float32)]
        ),
        scratch_shapes=[
            pltpu.VMEM((_E, _TOPK), jnp.float32),
            pltpu.VMEM((_E, 1), jnp.float32),
        ],
        interpret=interpret,
    )(x, W, triu)
    gates = outs[0:_TOPK]
    idxs = outs[_TOPK:2 * _TOPK]
    locw, counts, loss = outs[2 * _TOPK:]
    row_spec2 = pl.BlockSpec((1, _BS2), lambda i: (0, i))
    locs = pl.pallas_call(
        _offset_body,
        grid=(_NB2,),
        in_specs=(
            [pl.BlockSpec((_E, _TOPK), lambda i: (0, 0)),
             pl.BlockSpec((_TOPK, _BS2), lambda i: (0, i))]
            + [row_spec2] * _TOPK
        ),
        out_specs=[row_spec2] * _TOPK,
        out_shape=[jax.ShapeDtypeStruct((1, _S), jnp.int32)] * _TOPK,
        interpret=interpret,
    )(counts, locw, *idxs)
    return gates, idxs, locs, loss


def kernel(input, W):
    gates, idxs, locs, loss = _run(input, W)
    return (jnp.reshape(loss, ()),
            tuple(jnp.reshape(g, (_S,)) for g in gates),
            tuple(jnp.reshape(i, (_S,)) for i in idxs),
            tuple(jnp.reshape(l, (_S,)) for l in locs))


# speedup vs baseline: 5.5553x; 1.0698x over previous
import jax
import jax.numpy as jnp
from jax.experimental import pallas as pl
from jax.experimental.pallas import tpu as pltpu

_E = 64
_TOPK = 8
_D = 4096
_S = 8192
_BS = 1024
_NB = _S // _BS
_CS = 256
_NC = _BS // _CS
_BS2 = 2048
_NB2 = _S // _BS2
_EPS = float(jnp.finfo(jnp.float32).eps)


def _router_body(x_ref, w_ref, triu_ref, *refs):
    gates_refs = refs[0:_TOPK]
    idx_refs = refs[_TOPK:2 * _TOPK]
    locw_ref, counts_ref, loss_ref, carry_ref, me_acc = refs[2 * _TOPK:]
    b = pl.program_id(0)

    @pl.when(b == 0)
    def _():
        carry_ref[...] = jnp.zeros_like(carry_ref)
        me_acc[...] = jnp.zeros_like(me_acc)

    logits = jax.lax.dot_general(
        w_ref[...], x_ref[...], (((1,), (1,)), ((), ())),
        preferred_element_type=jnp.float32)

    iota_e = jax.lax.broadcasted_iota(jnp.int32, (_E, _BS), 0)

    cur = logits
    val_rows, idx_rows, sels = [], [], []
    for _ in range(_TOPK):
        m = jnp.max(cur, axis=0, keepdims=True)
        ik = jnp.min(jnp.where(cur == m, iota_e, _E), axis=0, keepdims=True)
        sel = iota_e == ik
        cur = jnp.where(sel, -jnp.inf, cur)
        val_rows.append(m)
        idx_rows.append(ik)
        sels.append(sel)

    maxv = val_rows[0]
    expl = jnp.exp(logits - maxv)
    inv = 1.0 / jnp.sum(expl, axis=0, keepdims=True)
    gate_rows = [jnp.exp(v - maxv) * inv for v in val_rows]
    denom = gate_rows[0]
    for g in gate_rows[1:]:
        denom = denom + g
    inv_denom = 1.0 / jnp.maximum(denom, _EPS)
    for k in range(_TOPK):
        gates_refs[k][...] = gate_rows[k] * inv_denom
        idx_refs[k][...] = idx_rows[k]

    sel_all = jnp.concatenate(sels, axis=0).astype(jnp.bfloat16)
    csum_chunks = []
    off = None
    for c in range(_NC):
        part = jax.lax.dot_general(
            sel_all[:, c * _CS:(c + 1) * _CS], triu_ref[...],
            (((1,), (0,)), ((), ())), preferred_element_type=jnp.float32)
        if off is not None:
            part = part + off
        off = part[:, _CS - 1:_CS]
        csum_chunks.append(part)
    csum_all = jnp.concatenate(csum_chunks, axis=1)
    locw_rows, cnt_cols = [], []
    for k in range(_TOPK):
        sel = sels[k]
        csum = csum_all[k * _E:(k + 1) * _E, :]
        carry_k = carry_ref[:, k:k + 1]
        loc_f = jnp.sum(jnp.where(sel, csum - 1.0 + carry_k, 0.0),
                        axis=0, keepdims=True)
        locw_rows.append(loc_f.astype(jnp.int32))
        cnt_cols.append(csum[:, _BS - 1:_BS])
    locw_ref[...] = jnp.concatenate(locw_rows, axis=0)
    carry_ref[...] = carry_ref[...] + jnp.concatenate(cnt_cols, axis=1)

    me_acc[...] = me_acc[...] + jnp.sum(expl * inv, axis=1, keepdims=True)

    @pl.when(b == _NB - 1)
    def _():
        counts_ref[...] = carry_ref[...]
        loss_ref[...] = (jnp.sum(me_acc[...] * carry_ref[:, 0:1],
                                 keepdims=True) * (_E / (_S * _S)))


def _offset_body(counts_ref, locw_ref, *refs):
    idx_refs = refs[0:_TOPK]
    loc_refs = refs[_TOPK:]
    counts = counts_ref[...]
    iota_e = jax.lax.broadcasted_iota(jnp.int32, (_E, _BS2), 0)
    loc_refs[0][...] = locw_ref[0:1, :]
    acc = counts[:, 0:1]
    for k in range(1, _TOPK):
        eq = iota_e == idx_refs[k][...]
        off_tok = jnp.sum(jnp.where(eq, acc, 0.0), axis=0, keepdims=True)
        loc_refs[k][...] = locw_ref[k:k + 1, :] + off_tok.astype(jnp.int32)
        if k < _TOPK - 1:
            acc = acc + counts[:, k:k + 1]


def _run(x, W, interpret=False):
    triu = jnp.triu(jnp.ones((_CS, _CS), jnp.bfloat16))
    row_spec = pl.BlockSpec((1, _BS), lambda i: (0, i))
    outs = pl.pallas_call(
        _router_body,
        grid=(_NB,),
        in_specs=[
            pl.BlockSpec((_BS, _D), lambda i: (i, 0)),
            pl.BlockSpec((_E, _D), lambda i: (0, 0)),
            pl.BlockSpec((_CS, _CS), lambda i: (0, 0)),
        ],
        out_specs=(
            [row_spec] * (2 * _TOPK)
            + [pl.BlockSpec((_TOPK, _BS), lambda i: (0, i)),
               pl.BlockSpec((_E, _TOPK), lambda i: (0, 0)),
               pl.BlockSpec((1, 1), lambda i: (0, 0))]
        ),
        out_shape=(
            [jax.ShapeDtypeStruct((1, _S), jnp.float32)] * _TOPK
            + [jax.ShapeDtypeStruct((1, _S), jnp.int32)] * _TOPK
            + [jax.ShapeDtypeStruct((_TOPK, _S), jnp.int32),
               jax.ShapeDtypeStruct((_E, _TOPK), jnp.float32),
               jax.ShapeDtypeStruct((1, 1), jnp.float32)]
        ),
        scratch_shapes=[
            pltpu.VMEM((_E, _TOPK), jnp.float32),
            pltpu.VMEM((_E, 1), jnp.float32),
        ],
        interpret=interpret,
    )(x, W, triu)
    gates = outs[0:_TOPK]
    idxs = outs[_TOPK:2 * _TOPK]
    locw, counts, loss = outs[2 * _TOPK:]
    row_spec2 = pl.BlockSpec((1, _BS2), lambda i: (0, i))
    locs = pl.pallas_call(
        _offset_body,
        grid=(_NB2,),
        in_specs=(
            [pl.BlockSpec((_E, _TOPK), lambda i: (0, 0)),
             pl.BlockSpec((_TOPK, _BS2), lambda i: (0, i))]
            + [row_spec2] * _TOPK
        ),
        out_specs=[row_spec2] * _TOPK,
        out_shape=[jax.ShapeDtypeStruct((1, _S), jnp.int32)] * _TOPK,
        interpret=interpret,
    )(counts, locw, *idxs)
    return gates, idxs, locs, loss


def kernel(input, W):
    gates, idxs, locs, loss = _run(input, W)
    return (jnp.reshape(loss, ()),
            tuple(jnp.reshape(g, (_S,)) for g in gates),
            tuple(jnp.reshape(i, (_S,)) for i in idxs),
            tuple(jnp.reshape(l, (_S,)) for l in locs))
